# baseline (speedup 1.0000x reference)
import functools

import jax
import jax.numpy as jnp
import numpy as np
from jax import lax
from jax.experimental import pallas as pl
from jax.experimental.pallas import tpu as pltpu
from jax.experimental.pallas import tpu_sc as plsc

N_NODES = 10000
N_EDGES = 320000
FDIM = 128

NC = 2
NS = 16
NW = NC * NS

NPAD = 10240
SLICE = NPAD // NS
CHUNK = 10240
LAST = N_EDGES - (NW - 1) * CHUNK
HALF = CHUNK // 2
NPIPE = 4
STEP = CHUNK // NPIPE
NSPLIT = 9984

_MESH = plsc.VectorSubcoreMesh(core_axis_name="c", subcore_axis_name="s")

_ONES = np.ones((HALF,), np.float32)


def _rsqrt_vec(d):
    m = d
    scale = jnp.full((16,), 1.0, jnp.float32)
    for _ in range(9):
        big = m > 4.0
        m = jnp.where(big, m * 0.25, m)
        scale = jnp.where(big, scale * 0.5, scale)
    y = scale * 0.7
    half_d = 0.5 * d
    for _ in range(4):
        y = y * (1.5 - half_d * y * y)
    return y


def _deg_body(edge_hbm, ones_hbm, out0_hbm, out1_hbm,
              dst_a, dst_b, ones_v, zslice_v, deg_sp, sem_a, sem_b):
    cid = lax.axis_index("c")
    sid = lax.axis_index("s")
    wid = cid * NS + sid
    base = pl.multiple_of(wid * CHUNK, 128)

    pltpu.sync_copy(ones_hbm, ones_v)

    def zbody(j, carry):
        zslice_v[pl.ds(j * 16, 16)] = jnp.full((16,), 0.0, jnp.float32)
        return carry

    lax.fori_loop(0, SLICE // 16, zbody, 0)
    pltpu.sync_copy(zslice_v, deg_sp.at[pl.ds(sid * SLICE, SLICE)])

    @pl.when(wid < NW - 1)
    def _():
        pltpu.sync_copy(edge_hbm.at[1, pl.ds(base, HALF)], dst_a)
        pltpu.sync_copy(edge_hbm.at[1, pl.ds(base + HALF, HALF)], dst_b)

    @pl.when(wid == NW - 1)
    def _():
        pltpu.sync_copy(edge_hbm.at[1, pl.ds(base, LAST)],
                        dst_a.at[pl.ds(0, LAST)])

    plsc.subcore_barrier()

    @pl.when(wid < NW - 1)
    def _():
        cp_a = pltpu.async_copy(ones_v, deg_sp.at[dst_a], sem_a, add=True)
        cp_b = pltpu.async_copy(ones_v, deg_sp.at[dst_b], sem_b, add=True)
        cp_a.wait()
        cp_b.wait()

    @pl.when(wid == NW - 1)
    def _():
        pltpu.sync_copy(ones_v.at[pl.ds(0, LAST)],
                        deg_sp.at[dst_a.at[pl.ds(0, LAST)]], add=True)

    plsc.subcore_barrier()

    @pl.when(sid == 0)
    def _():
        @pl.when(cid == 0)
        def _():
            pltpu.sync_copy(deg_sp, out0_hbm.at[0])

        @pl.when(cid == 1)
        def _():
            pltpu.sync_copy(deg_sp, out1_hbm.at[0])


_deg_kernel = functools.partial(
    pl.kernel,
    out_type=(jax.ShapeDtypeStruct((1, NPAD), jnp.float32),
              jax.ShapeDtypeStruct((1, NPAD), jnp.float32)),
    mesh=_MESH,
    scratch_types=[
        pltpu.VMEM((HALF,), jnp.int32),
        pltpu.VMEM((HALF,), jnp.int32),
        pltpu.VMEM((HALF,), jnp.float32),
        pltpu.VMEM((SLICE,), jnp.float32),
        pltpu.VMEM_SHARED((NPAD,), jnp.float32),
        pltpu.SemaphoreType.DMA,
        pltpu.SemaphoreType.DMA,
    ],
)(_deg_body)


def _s_body(edge_hbm, deg0_hbm, deg1_hbm, s0_hbm, s1_hbm,
            src0, src1, src2, src3, dst0, dst1, dst2, dst3,
            val0, val1, val2, val3, dbuf, dtile_v, dinv_sp, s_sp,
            gsem, ssem):
    cid = lax.axis_index("c")
    sid = lax.axis_index("s")
    wid = cid * NS + sid
    base = pl.multiple_of(wid * CHUNK, 128)
    nbase = sid * SLICE
    srcs = [src0, src1, src2, src3]
    dsts = [dst0, dst1, dst2, dst3]
    vals = [val0, val1, val2, val3]

    pltpu.sync_copy(deg0_hbm.at[0, pl.ds(nbase, SLICE)], dbuf.at[0])
    pltpu.sync_copy(deg1_hbm.at[0, pl.ds(nbase, SLICE)], dbuf.at[1])

    @pl.when(wid < NW - 1)
    def _():
        for k in range(NPIPE):
            pltpu.sync_copy(edge_hbm.at[0, pl.ds(base + k * STEP, STEP)],
                            srcs[k])
            pltpu.sync_copy(edge_hbm.at[1, pl.ds(base + k * STEP, STEP)],
                            dsts[k])

    @pl.when(wid == NW - 1)
    def _():
        pltpu.sync_copy(edge_hbm.at[0, pl.ds(base, LAST)], srcs[0])
        pltpu.sync_copy(edge_hbm.at[1, pl.ds(base, LAST)], dsts[0])

    def rs_body(j, carry):
        off = j * 16
        d = dbuf[0, pl.ds(off, 16)] + dbuf[1, pl.ds(off, 16)] + 1.0
        dtile_v[pl.ds(off, 16)] = _rsqrt_vec(d)
        return carry

    lax.fori_loop(0, SLICE // 16, rs_body, 0)
    pltpu.sync_copy(dtile_v, dinv_sp.at[pl.ds(nbase, SLICE)])

    def zbody(j, carry):
        dtile_v[pl.ds(j * 16, 16)] = jnp.full((16,), 0.0, jnp.float32)
        return carry

    lax.fori_loop(0, SLICE // 16, zbody, 0)
    pltpu.sync_copy(dtile_v, s_sp.at[pl.ds(nbase, SLICE)])

    plsc.subcore_barrier()

    @pl.when(wid < NW - 1)
    def _():
        g = pltpu.async_copy(dinv_sp.at[dsts[0]], vals[0], gsem)
        scs = []
        for k in range(NPIPE):
            g.wait()
            scs.append(pltpu.async_copy(vals[k], s_sp.at[srcs[k]], ssem,
                                        add=True))
            if k + 1 < NPIPE:
                g = pltpu.async_copy(dinv_sp.at[dsts[k + 1]], vals[k + 1],
                                     gsem)
        for sc in scs:
            sc.wait()

    @pl.when(wid == NW - 1)
    def _():
        pltpu.sync_copy(dinv_sp.at[dsts[0]], vals[0])
        pltpu.sync_copy(vals[0], s_sp.at[srcs[0]], add=True)

    plsc.subcore_barrier()

    @pl.when(sid == 0)
    def _():
        @pl.when(cid == 0)
        def _():
            pltpu.sync_copy(s_sp, s0_hbm.at[0])

        @pl.when(cid == 1)
        def _():
            pltpu.sync_copy(s_sp, s1_hbm.at[0])


_s_kernel = functools.partial(
    pl.kernel,
    out_type=(jax.ShapeDtypeStruct((1, NPAD), jnp.float32),
              jax.ShapeDtypeStruct((1, NPAD), jnp.float32)),
    mesh=_MESH,
    scratch_types=(
        [pltpu.VMEM((STEP,), jnp.int32)] * 8
        + [pltpu.VMEM((STEP,), jnp.float32)] * 4
        + [
            pltpu.VMEM((2, SLICE), jnp.float32),
            pltpu.VMEM((SLICE,), jnp.float32),
            pltpu.VMEM_SHARED((NPAD,), jnp.float32),
            pltpu.VMEM_SHARED((NPAD,), jnp.float32),
            pltpu.SemaphoreType.DMA,
            pltpu.SemaphoreType.DMA,
        ]
    ),
)(_s_body)


def _out_body(d0_ref, d1_ref, s0_ref, s1_ref, x_ref, w_ref, b_ref, out_ref):
    deg = d0_ref[...] + d1_ref[...] + 1.0
    idx = lax.broadcasted_iota(jnp.int32, (1, NPAD), 1)
    dinv = jnp.where(idx < N_NODES, lax.rsqrt(deg), 0.0)
    coef = dinv * (s0_ref[...] + s1_ref[...] + dinv)
    acc = jnp.dot(coef[:, :NSPLIT], x_ref[:NSPLIT, :],
                  preferred_element_type=jnp.float32)
    acc = acc + jnp.dot(coef[:, NSPLIT:N_NODES], x_ref[NSPLIT:, :],
                        preferred_element_type=jnp.float32)
    out_ref[...] = (
        lax.dot_general(acc, w_ref[...], (((1,), (1,)), ((), ())),
                        preferred_element_type=jnp.float32)
        + float(N_NODES) * b_ref[...]
    )


def _out_call(d0, d1, s0, s1, x, w, b2):
    return pl.pallas_call(
        _out_body,
        out_shape=jax.ShapeDtypeStruct((1, FDIM), jnp.float32),
    )(d0, d1, s0, s1, x, w, b2)


@jax.jit
def kernel(x, edge_index, W, b):
    ones = jnp.asarray(_ONES)
    deg0, deg1 = _deg_kernel(edge_index, ones)
    s0, s1 = _s_kernel(edge_index, deg0, deg1)
    return _out_call(deg0, deg1, s0, s1, x, W, b.reshape(1, FDIM))


# speedup vs baseline: 1.0827x; 1.0827x over previous
import functools

import jax
import jax.numpy as jnp
import numpy as np
from jax import lax
from jax.experimental import pallas as pl
from jax.experimental.pallas import tpu as pltpu
from jax.experimental.pallas import tpu_sc as plsc

N_NODES = 10000
N_EDGES = 320000
FDIM = 128

NC = 2
NS = 16
NW = NC * NS

NPAD = 10240
SLICE = NPAD // NS
CHUNK = 10240
LAST = N_EDGES - (NW - 1) * CHUNK
NSPLIT = 9984

_MESH = plsc.VectorSubcoreMesh(core_axis_name="c", subcore_axis_name="s")

_ONES = np.ones((CHUNK,), np.float32)


def _rsqrt_vec(d):
    m = d
    scale = jnp.full((16,), 1.0, jnp.float32)
    for _ in range(9):
        big = m > 4.0
        m = jnp.where(big, m * 0.25, m)
        scale = jnp.where(big, scale * 0.5, scale)
    y = scale * 0.7
    half_d = 0.5 * d
    for _ in range(5):
        y = y * (1.5 - half_d * y * y)
    return y


def _deg_body(edge_hbm, ones_hbm, out0_hbm, out1_hbm,
              dst_v, ones_v, zslice_v, deg_sp):
    cid = lax.axis_index("c")
    sid = lax.axis_index("s")
    wid = cid * NS + sid
    base = pl.multiple_of(wid * CHUNK, 128)

    pltpu.sync_copy(ones_hbm, ones_v)

    def zbody(j, carry):
        zslice_v[pl.ds(j * 16, 16)] = jnp.full((16,), 0.0, jnp.float32)
        return carry

    lax.fori_loop(0, SLICE // 16, zbody, 0)
    pltpu.sync_copy(zslice_v, deg_sp.at[pl.ds(sid * SLICE, SLICE)])

    @pl.when(wid < NW - 1)
    def _():
        pltpu.sync_copy(edge_hbm.at[1, pl.ds(base, CHUNK)], dst_v)

    @pl.when(wid == NW - 1)
    def _():
        pltpu.sync_copy(edge_hbm.at[1, pl.ds(base, LAST)],
                        dst_v.at[pl.ds(0, LAST)])

    plsc.subcore_barrier()

    @pl.when(wid < NW - 1)
    def _():
        pltpu.sync_copy(ones_v, deg_sp.at[dst_v], add=True)

    @pl.when(wid == NW - 1)
    def _():
        pltpu.sync_copy(ones_v.at[pl.ds(0, LAST)],
                        deg_sp.at[dst_v.at[pl.ds(0, LAST)]], add=True)

    plsc.subcore_barrier()

    @pl.when(sid == 0)
    def _():
        @pl.when(cid == 0)
        def _():
            pltpu.sync_copy(deg_sp, out0_hbm.at[0])

        @pl.when(cid == 1)
        def _():
            pltpu.sync_copy(deg_sp, out1_hbm.at[0])


_deg_kernel = functools.partial(
    pl.kernel,
    out_type=(jax.ShapeDtypeStruct((1, NPAD), jnp.float32),
              jax.ShapeDtypeStruct((1, NPAD), jnp.float32)),
    mesh=_MESH,
    scratch_types=[
        pltpu.VMEM((CHUNK,), jnp.int32),
        pltpu.VMEM((CHUNK,), jnp.float32),
        pltpu.VMEM((SLICE,), jnp.float32),
        pltpu.VMEM_SHARED((NPAD,), jnp.float32),
    ],
)(_deg_body)


def _s_body(edge_hbm, deg0_hbm, deg1_hbm, s0_hbm, s1_hbm,
            src_v, dst_v, vals_v, dbuf, dtile_v, dinv_sp, s_sp):
    cid = lax.axis_index("c")
    sid = lax.axis_index("s")
    wid = cid * NS + sid
    base = pl.multiple_of(wid * CHUNK, 128)
    nbase = sid * SLICE

    @pl.when(wid < NW - 1)
    def _():
        pltpu.sync_copy(edge_hbm.at[0, pl.ds(base, CHUNK)], src_v)
        pltpu.sync_copy(edge_hbm.at[1, pl.ds(base, CHUNK)], dst_v)

    @pl.when(wid == NW - 1)
    def _():
        pltpu.sync_copy(edge_hbm.at[0, pl.ds(base, LAST)],
                        src_v.at[pl.ds(0, LAST)])
        pltpu.sync_copy(edge_hbm.at[1, pl.ds(base, LAST)],
                        dst_v.at[pl.ds(0, LAST)])

    pltpu.sync_copy(deg0_hbm.at[0, pl.ds(nbase, SLICE)], dbuf.at[0])
    pltpu.sync_copy(deg1_hbm.at[0, pl.ds(nbase, SLICE)], dbuf.at[1])

    def rs_body(j, carry):
        off = j * 16
        d = dbuf[0, pl.ds(off, 16)] + dbuf[1, pl.ds(off, 16)] + 1.0
        dtile_v[pl.ds(off, 16)] = _rsqrt_vec(d)
        return carry

    lax.fori_loop(0, SLICE // 16, rs_body, 0)
    pltpu.sync_copy(dtile_v, dinv_sp.at[pl.ds(nbase, SLICE)])

    def zbody(j, carry):
        dtile_v[pl.ds(j * 16, 16)] = jnp.full((16,), 0.0, jnp.float32)
        return carry

    lax.fori_loop(0, SLICE // 16, zbody, 0)
    pltpu.sync_copy(dtile_v, s_sp.at[pl.ds(nbase, SLICE)])

    plsc.subcore_barrier()

    @pl.when(wid < NW - 1)
    def _():
        pltpu.sync_copy(dinv_sp.at[dst_v], vals_v)
        pltpu.sync_copy(vals_v, s_sp.at[src_v], add=True)

    @pl.when(wid == NW - 1)
    def _():
        pltpu.sync_copy(dinv_sp.at[dst_v.at[pl.ds(0, LAST)]],
                        vals_v.at[pl.ds(0, LAST)])
        pltpu.sync_copy(vals_v.at[pl.ds(0, LAST)],
                        s_sp.at[src_v.at[pl.ds(0, LAST)]], add=True)

    plsc.subcore_barrier()

    @pl.when(sid == 0)
    def _():
        @pl.when(cid == 0)
        def _():
            pltpu.sync_copy(s_sp, s0_hbm.at[0])

        @pl.when(cid == 1)
        def _():
            pltpu.sync_copy(s_sp, s1_hbm.at[0])


_s_kernel = functools.partial(
    pl.kernel,
    out_type=(jax.ShapeDtypeStruct((1, NPAD), jnp.float32),
              jax.ShapeDtypeStruct((1, NPAD), jnp.float32)),
    mesh=_MESH,
    scratch_types=[
        pltpu.VMEM((CHUNK,), jnp.int32),
        pltpu.VMEM((CHUNK,), jnp.int32),
        pltpu.VMEM((CHUNK,), jnp.float32),
        pltpu.VMEM((2, SLICE), jnp.float32),
        pltpu.VMEM((SLICE,), jnp.float32),
        pltpu.VMEM_SHARED((NPAD,), jnp.float32),
        pltpu.VMEM_SHARED((NPAD,), jnp.float32),
    ],
)(_s_body)


def _out_body(d0_ref, d1_ref, s0_ref, s1_ref, x_ref, w_ref, b_ref, out_ref):
    deg = d0_ref[...] + d1_ref[...] + 1.0
    idx = lax.broadcasted_iota(jnp.int32, (1, NPAD), 1)
    dinv = jnp.where(idx < N_NODES, lax.rsqrt(deg), 0.0)
    coef = dinv * (s0_ref[...] + s1_ref[...] + dinv)
    acc = jnp.dot(coef[:, :NSPLIT], x_ref[:NSPLIT, :],
                  preferred_element_type=jnp.float32)
    acc = acc + jnp.dot(coef[:, NSPLIT:N_NODES], x_ref[NSPLIT:, :],
                        preferred_element_type=jnp.float32)
    out_ref[...] = (
        lax.dot_general(acc, w_ref[...], (((1,), (1,)), ((), ())),
                        preferred_element_type=jnp.float32)
        + float(N_NODES) * b_ref[...]
    )


def _out_call(d0, d1, s0, s1, x, w, b2):
    return pl.pallas_call(
        _out_body,
        out_shape=jax.ShapeDtypeStruct((1, FDIM), jnp.float32),
    )(d0, d1, s0, s1, x, w, b2)


@jax.jit
def kernel(x, edge_index, W, b):
    ones = jnp.asarray(_ONES)
    deg0, deg1 = _deg_kernel(edge_index, ones)
    s0, s1 = _s_kernel(edge_index, deg0, deg1)
    return _out_call(deg0, deg1, s0, s1, x, W, b.reshape(1, FDIM))
